# baseline (speedup 1.0000x reference)
import functools

import jax
import jax.numpy as jnp
from jax import lax
from jax.experimental import pallas as pl
from jax.experimental.pallas import tpu as pltpu
from jax.experimental.pallas import tpu_sc as plsc

_ANCHORS = (
    (1.3221, 1.73145),
    (3.19275, 4.00944),
    (5.05587, 8.09892),
    (9.47112, 4.84053),
    (11.2364, 10.0071),
)
_NUM_CLASSES = 20
_LAMBDA_COORD = 5.0
_LAMBDA_OBJ = 1.0
_LAMBDA_NOOBJ = 0.5
_LAMBDA_CLS = 1.0
_B, _H, _W = 64, 19, 19
_A = len(_ANCHORS)
_HW = _H * _W
_C = 5 + _NUM_CLASSES

_NC, _NS, _L = 2, 16, 16
_NW = _NC * _NS
_BPW = _B // _NW
_NG = -(-_HW // _L)
_TAIL_OFF = _HW - _L
_TAIL_DUP = _NG * _L - _HW
_LN2 = 0.6931471805599453


def _ln(x):
    bits = lax.bitcast_convert_type(x, jnp.int32)
    e = ((bits >> 23) & 255) - 127
    m = lax.bitcast_convert_type((bits & 0x007FFFFF) | 0x3F800000, jnp.float32)
    big = m > 1.4142135381698608
    m = jnp.where(big, m * 0.5, m)
    ef = e.astype(jnp.float32) + jnp.where(big, 1.0, 0.0)
    t = (m - 1.0) / (m + 1.0)
    t2 = t * t
    p = 2.0 * t * (
        1.0 + t2 * (1.0 / 3.0 + t2 * (1.0 / 5.0 + t2 * (1.0 / 7.0 + t2 * (1.0 / 9.0))))
    )
    return ef * _LN2 + p


def _sigmoid(x):
    return 1.0 / (1.0 + jnp.exp(-x))


def _sc_body(pred_hbm, tgt_hbm, out_hbm, pred_v, tgt_v, acc_v):
    wid = lax.axis_index("s") * _NC + lax.axis_index("c")
    b0 = wid * _BPW
    pltpu.sync_copy(pred_hbm.at[pl.ds(b0, _BPW)], pred_v)
    pltpu.sync_copy(tgt_hbm.at[pl.ds(b0, _BPW)], tgt_v)

    iota = lax.iota(jnp.int32, _L)
    zero = jnp.zeros((_L,), jnp.float32)

    def run_batch(bb, accs):
        def group(off, ok, accs):
            box_a, conf_a, noobj_a, cls_a = accs

            def pch(c):
                return pred_v[bb, c, pl.ds(off, _L)]

            def tch(c):
                return tgt_v[bb, c, pl.ds(off, _L)]

            gconf = tch(20)
            gx = tch(21)
            gy = tch(22)
            gw = tch(23)
            gh = tch(24)
            b2x1 = gx - gw / 2.0
            b2y1 = gy - gh / 2.0
            b2x2 = gx + gw / 2.0
            b2y2 = gy + gh / 2.0
            a2 = (b2x2 - b2x1) * (b2y2 - b2y1)

            best_iou = best_idx = None
            sel_conf = sel_x = sel_y = sel_w = sel_h = None
            for a in range(_A):
                base = a * _C
                aw, ah = _ANCHORS[a]
                tconf = _sigmoid(pch(base + 20))
                px = _sigmoid(pch(base + 21))
                py = _sigmoid(pch(base + 22))
                pw = jnp.exp(pch(base + 23)) * aw
                ph = jnp.exp(pch(base + 24)) * ah
                b1x1 = px - pw / 2.0
                b1y1 = py - ph / 2.0
                b1x2 = px + pw / 2.0
                b1y2 = py + ph / 2.0
                ix1 = jnp.maximum(b1x1, b2x1)
                iy1 = jnp.maximum(b1y1, b2y1)
                ix2 = jnp.minimum(b1x2, b2x2)
                iy2 = jnp.minimum(b1y2, b2y2)
                iw = jnp.maximum(ix2 - ix1, 0.0)
                ih = jnp.maximum(iy2 - iy1, 0.0)
                inter = iw * ih
                a1 = (b1x2 - b1x1) * (b1y2 - b1y1)
                union = a1 + a2 - inter
                iou = inter / (union + 1e-10)
                if a == 0:
                    best_iou = iou
                    best_idx = jnp.zeros((_L,), jnp.int32)
                    sel_conf, sel_x, sel_y, sel_w, sel_h = tconf, px, py, pw, ph
                else:
                    upd = iou > best_iou
                    best_iou = jnp.where(upd, iou, best_iou)
                    best_idx = jnp.where(upd, a, best_idx)
                    sel_conf = jnp.where(upd, tconf, sel_conf)
                    sel_x = jnp.where(upd, px, sel_x)
                    sel_y = jnp.where(upd, py, sel_y)
                    sel_w = jnp.where(upd, pw, sel_w)
                    sel_h = jnp.where(upd, ph, sel_h)

            masks = [
                jnp.where(best_idx == a, 1.0, 0.0) for a in range(_A)
            ]
            sel_logits = []
            for c in range(_NUM_CLASSES):
                sl = masks[0] * pch(c)
                for a in range(1, _A):
                    sl = sl + masks[a] * pch(a * _C + c)
                sel_logits.append(sl)
            m = sel_logits[0]
            for c in range(1, _NUM_CLASSES):
                m = jnp.maximum(m, sel_logits[c])
            s = jnp.exp(sel_logits[0] - m)
            pick = tch(0) * sel_logits[0]
            for c in range(1, _NUM_CLASSES):
                s = s + jnp.exp(sel_logits[c] - m)
                pick = pick + tch(c) * sel_logits[c]
            lse = m + _ln(s)

            obj = jnp.where(gconf != 0.0, 1.0, 0.0)
            noobj = jnp.where((1.0 - gconf) != 0.0, 1.0, 0.0)

            dx = sel_x - gx
            dy = sel_y - gy
            dw = sel_w - gw
            dh = sel_h - gh
            box_t = obj * (dx * dx + dy * dy + dw * dw + dh * dh)
            dc = sel_conf - gconf
            conf_t = obj * (dc * dc)
            noobj_t = noobj * sel_conf * sel_conf
            cls_t = obj * (lse - pick)
            return (
                box_a + jnp.where(ok, box_t, zero),
                conf_a + jnp.where(ok, conf_t, zero),
                noobj_a + jnp.where(ok, noobj_t, zero),
                cls_a + jnp.where(ok, cls_t, zero),
            )

        def aligned(g, accs):
            return group(pl.multiple_of(g * _L, _L), iota >= 0, accs)

        accs = lax.fori_loop(0, _NG - 1, aligned, accs)
        return group(_TAIL_OFF, iota >= _TAIL_DUP, accs)

    accs = (zero, zero, zero, zero)
    for bb in range(_BPW):
        accs = run_batch(bb, accs)

    scales = (
        _LAMBDA_COORD / _B,
        _LAMBDA_OBJ / _B,
        _LAMBDA_NOOBJ / _B,
        _LAMBDA_CLS / _B,
    )
    for i in range(4):
        acc_v[i, :] = accs[i] * scales[i]
    for i in range(4):
        pltpu.sync_copy(acc_v.at[i], out_hbm.at[i, wid])


_sc_loss_kernel = functools.partial(
    pl.kernel,
    out_type=jax.ShapeDtypeStruct((4, _NW, _L), jnp.float32),
    mesh=plsc.VectorSubcoreMesh(core_axis_name="c", subcore_axis_name="s"),
    scratch_types=[
        pltpu.VMEM((_BPW, _A * _C, _HW), jnp.float32),
        pltpu.VMEM((_BPW, _C, _HW), jnp.float32),
        pltpu.VMEM((4, _L), jnp.float32),
    ],
)(_sc_body)


def kernel(prediction, target):
    pred = prediction.reshape(_B, _A * _C, _HW)
    tgt = jnp.transpose(target.reshape(_B, _HW, _C), (0, 2, 1))
    parts = _sc_loss_kernel(pred, tgt)
    s = jnp.sum(parts, axis=(1, 2))
    return (s[0], s[1], s[2], s[3])
